# baseline (speedup 1.0000x reference)
import jax
import jax.numpy as jnp
from jax.experimental import pallas as pl
from jax.experimental.pallas import tpu as pltpu

_LOG2E = 1.4426950408889634


def _hist_kernel(x_ref, e_ref, o_ref):
    x = x_ref[...]
    e = e_ref[...]

    mu_in = (e[:, :-1] + e[:, 1:]) * 0.5
    mus = jnp.concatenate([e[:, :1], mu_in], axis=1)
    sig0 = (e[:, :1] - e[:, 1:2]) / 3.0
    sig_in = (e[:, :-1] - e[:, 1:]) / 3.0
    sigs = jnp.concatenate([sig0, sig_in], axis=1) + 1e-6
    inv = 1.0 / sigs

    n_e = e.shape[1]
    cols = []
    for i in range(n_e):
        z = (x - mus[:, i:i + 1]) * inv[:, i:i + 1]
        arg = (z * z) * (-0.5 * _LOG2E)
        cols.append(jnp.sum(jnp.exp2(arg), axis=1, keepdims=True))
    t = (e[:, n_e - 1:n_e] - x) * (20.0 * _LOG2E)
    cols.append(jnp.sum(1.0 / (1.0 + jnp.exp2(t)), axis=1, keepdims=True))
    o_ref[...] = jnp.concatenate(cols, axis=1)


def kernel(x, hist_edges):
    bt, c, h, w = x.shape
    n_e = hist_edges.shape[1]
    hw = h * w
    x2d = x.reshape(bt * c, hw)

    out = pl.pallas_call(
        _hist_kernel,
        grid=(bt,),
        in_specs=[
            pl.BlockSpec((c, hw), lambda i: (i, 0)),
            pl.BlockSpec((c, n_e), lambda i: (0, 0)),
        ],
        out_specs=pl.BlockSpec((c, n_e + 1), lambda i: (i, 0)),
        out_shape=jax.ShapeDtypeStruct((bt * c, n_e + 1), x.dtype),
        compiler_params=pltpu.CompilerParams(
            dimension_semantics=("parallel",),
        ),
    )(x2d, hist_edges)
    return out.reshape(bt, c, n_e + 1)


# speedup vs baseline: 2.4986x; 2.4986x over previous
import jax
import jax.numpy as jnp
from jax.experimental import pallas as pl
from jax.experimental.pallas import tpu as pltpu

_LOG2E = 1.4426950408889634


def _hist_kernel(x_ref, e_ref, o_ref):
    x = x_ref[0]
    e = e_ref[...]

    n_e = e.shape[0]
    rows = []
    for i in range(n_e):
        if i == 0:
            mu = e[0:1]
            sig = (e[0:1] - e[1:2]) * (1.0 / 3.0)
        else:
            mu = (e[i - 1:i] + e[i:i + 1]) * 0.5
            sig = (e[i - 1:i] - e[i:i + 1]) * (1.0 / 3.0)
        alpha = 1.0 / (sig + 1e-6)
        beta = alpha * (-0.5 * _LOG2E)
        d = x - mu
        rows.append(jnp.sum(jnp.exp2((d * alpha) * (d * beta)),
                            axis=0, keepdims=True))
    t = (e[n_e - 1:n_e] - x) * (20.0 * _LOG2E)
    rows.append(jnp.sum(1.0 / (1.0 + jnp.exp2(t)), axis=0, keepdims=True))
    o_ref[0] = jnp.concatenate(rows, axis=0)


def kernel(x, hist_edges):
    bt, c, h, w = x.shape
    n_e = hist_edges.shape[1]
    hw = h * w
    xp = jnp.transpose(x.reshape(bt, c, hw), (0, 2, 1))
    et = hist_edges.T

    out = pl.pallas_call(
        _hist_kernel,
        grid=(bt,),
        in_specs=[
            pl.BlockSpec((1, hw, c), lambda i: (i, 0, 0)),
            pl.BlockSpec((n_e, c), lambda i: (0, 0)),
        ],
        out_specs=pl.BlockSpec((1, n_e + 1, c), lambda i: (i, 0, 0)),
        out_shape=jax.ShapeDtypeStruct((bt, n_e + 1, c), x.dtype),
        compiler_params=pltpu.CompilerParams(
            dimension_semantics=("parallel",),
        ),
    )(xp, et)
    return jnp.transpose(out, (0, 2, 1))
